# Initial kernel scaffold; baseline (speedup 1.0000x reference)
import jax
import jax.numpy as jnp
from jax.experimental import pallas as pl


def kernel(x, edge_index, batch, emb_table, W0, b0, W1, b1, W2, b2, W3, b3, Wc1, bc1, Wc2, bc2):
    raise NotImplementedError("write your pallas kernel here")


# speedup vs baseline: 17.8707x; 17.8707x over previous
import functools

import jax
import jax.numpy as jnp
from jax import lax
from jax.experimental import pallas as pl
from jax.experimental.pallas import tpu as pltpu
from jax.experimental.pallas import tpu_sc as plsc

N = 10000
E = 640000
NPAD = 10240
NW = 32
EB = 125
NBLK = 160
NCH = 4
NBI = NBLK // NCH
RPT = NPAD // 16
XB = 80
XNB = 4
HID = 128
NG = 64
NL = 2


def _sc_precompute_body(dst_hbm, x_hbm, emb_hbm, zdeg_hbm, ones_hbm,
                        deg_out, h0_out, dst_v, ones_v, xv, erows, deg_sh, sem):
    c = lax.axis_index("c")
    s = lax.axis_index("s")
    w = s * 2 + c

    pltpu.sync_copy(dst_hbm.at[w], dst_v)
    pltpu.sync_copy(ones_hbm, ones_v)

    @pl.when(s == 0)
    def _():
        pltpu.sync_copy(zdeg_hbm, deg_sh)

    plsc.subcore_barrier()

    for jc in range(NCH):
        def deg_body(j, carry, jc=jc):
            pltpu.sync_copy(ones_v, deg_sh.at[dst_v.at[jc, j]], add=True)
            return carry

        lax.fori_loop(0, NBI, deg_body, 0)
    plsc.subcore_barrier()

    @pl.when(s == 0)
    def _():
        pltpu.sync_copy(deg_sh, deg_out.at[c])

    pltpu.sync_copy(x_hbm.at[w], xv)
    for jb in range(XNB):
        pltpu.async_copy(emb_hbm.at[xv.at[jb]], erows, sem).wait()
        pltpu.sync_copy(erows, h0_out.at[pl.ds(w * (XNB * XB) + jb * XB, XB)])


def _sc_propagate_body(hp_hbm, src_hbm, dst_hbm, zblk_hbm,
                       s_out, src_v, dst_v, rows_v, acc_sh, sem):
    c = lax.axis_index("c")
    s = lax.axis_index("s")
    w = s * 2 + c

    pltpu.sync_copy(zblk_hbm, acc_sh.at[pl.ds(s * RPT, RPT)])
    plsc.subcore_barrier()

    def body(j, carry):
        pltpu.async_copy(hp_hbm.at[src_v.at[j]], rows_v, sem).wait()
        pltpu.sync_copy(rows_v, acc_sh.at[dst_v.at[j]], add=True)
        return carry

    for jc in range(NCH):
        pltpu.sync_copy(src_hbm.at[w, jc], src_v)
        pltpu.sync_copy(dst_hbm.at[w, jc], dst_v)
        lax.fori_loop(0, NBI, body, 0)
    plsc.subcore_barrier()
    pltpu.sync_copy(acc_sh.at[pl.ds(s * RPT, RPT)],
                    s_out.at[c, pl.ds(s * RPT, RPT)])


@functools.lru_cache(maxsize=None)
def _sc_kernels():
    mesh = plsc.VectorSubcoreMesh(core_axis_name="c", subcore_axis_name="s")
    precompute = functools.partial(
        pl.kernel,
        out_type=(
            jax.ShapeDtypeStruct((2, NPAD), jnp.float32),
            jax.ShapeDtypeStruct((NPAD, HID), jnp.float32),
        ),
        mesh=mesh,
        scratch_types=[
            pltpu.VMEM((NCH, NBI, EB), jnp.int32),
            pltpu.VMEM((EB,), jnp.float32),
            pltpu.VMEM((XNB, XB), jnp.int32),
            pltpu.VMEM((XB, HID), jnp.float32),
            pltpu.VMEM_SHARED((NPAD,), jnp.float32),
            pltpu.SemaphoreType.DMA,
        ],
    )(_sc_precompute_body)
    propagate = functools.partial(
        pl.kernel,
        out_type=jax.ShapeDtypeStruct((2, NPAD, HID), jnp.float32),
        mesh=mesh,
        scratch_types=[
            pltpu.VMEM((NBI, EB), jnp.int32),
            pltpu.VMEM((NBI, EB), jnp.int32),
            pltpu.VMEM((EB, HID), jnp.float32),
            pltpu.VMEM_SHARED((NPAD, HID), jnp.float32),
            pltpu.SemaphoreType.DMA,
        ],
    )(_sc_propagate_body)
    return precompute, propagate


def _t0_body(deg_ref, h0_ref, w_ref, disb_ref, hp_ref):
    deg = deg_ref[0, :] + deg_ref[1, :] + 1.0
    dis = lax.rsqrt(jnp.maximum(deg, 1.0))
    disb = jnp.broadcast_to(dis[:, None], (NPAD, HID))
    disb_ref[...] = disb
    hp_ref[...] = disb * jnp.dot(h0_ref[...], w_ref[...],
                                 preferred_element_type=jnp.float32)


def _layer_body(s_ref, hp_ref, disb_ref, b_ref, w_ref, out_ref):
    tot = s_ref[0] + s_ref[1] + hp_ref[...]
    h = jnp.maximum(tot * disb_ref[...] + b_ref[...][None, :], 0.0)
    out_ref[...] = disb_ref[...] * jnp.dot(h, w_ref[...],
                                           preferred_element_type=jnp.float32)


def _final_body(s_ref, hp_ref, disb_ref, b_ref, batch_ref, batchb_ref,
                wc1_ref, bc1_ref, wc2_ref, bc2_ref, out_ref):
    tot = s_ref[0] + s_ref[1] + hp_ref[...]
    h = jnp.maximum(tot * disb_ref[...] + b_ref[...][None, :], 0.0)

    bt = batch_ref[...]
    gi = lax.broadcasted_iota(jnp.int32, (NG, NPAD), 0)
    m = (gi == bt[None, :]).astype(jnp.float32)
    counts = jnp.dot(m, jnp.ones((NPAD, 1), jnp.float32),
                     preferred_element_type=jnp.float32)
    sums = jnp.dot(m, h, preferred_element_type=jnp.float32)
    mean = sums / jnp.maximum(counts, 1.0)

    btb = batchb_ref[...]
    parts = [jnp.max(jnp.where(btb == g, h, 0.0), axis=0, keepdims=True)
             for g in range(NG)]
    maxs = jnp.concatenate(parts, axis=0)

    g1 = (jnp.dot(mean, wc1_ref[:HID, :], preferred_element_type=jnp.float32)
          + jnp.dot(maxs, wc1_ref[HID:, :], preferred_element_type=jnp.float32)
          + bc1_ref[...][None, :])
    g1 = jnp.maximum(g1, 0.0)
    out_ref[...] = (jnp.dot(g1, wc2_ref[...], preferred_element_type=jnp.float32)
                    + bc2_ref[...][None, :])


def kernel(x, edge_index, batch, emb_table,
           W0, b0, W1, b1, W2, b2, W3, b3, Wc1, bc1, Wc2, bc2):
    f32 = jnp.float32
    src = edge_index[0].astype(jnp.int32).reshape(NW, NCH, NBI, EB)
    dst = edge_index[1].astype(jnp.int32).reshape(NW, NCH, NBI, EB)
    xw = jnp.pad(x[:, 0].astype(jnp.int32), (0, NPAD - N)).reshape(NW, XNB, XB)
    batch_pad = jnp.pad(batch.astype(jnp.int32), (0, NPAD - N),
                        constant_values=NG)
    batch_b = jnp.broadcast_to(batch_pad[:, None], (NPAD, HID))

    zdeg = jnp.zeros((NPAD,), f32)
    zblk = jnp.zeros((RPT, HID), f32)
    onesb = jnp.ones((EB,), f32)

    _sc_precompute, _sc_propagate = _sc_kernels()
    deg2, h0 = _sc_precompute(dst, xw, emb_table, zdeg, onesb)

    disb, hp = pl.pallas_call(
        _t0_body,
        out_shape=(jax.ShapeDtypeStruct((NPAD, HID), f32),
                   jax.ShapeDtypeStruct((NPAD, HID), f32)),
    )(deg2, h0, W0)

    for (b_prev, w_next) in ((b0, W1), (b1, W2), (b2, W3)):
        s_parts = _sc_propagate(hp, src, dst, zblk)
        hp = pl.pallas_call(
            _layer_body,
            out_shape=jax.ShapeDtypeStruct((NPAD, HID), f32),
        )(s_parts, hp, disb, b_prev, w_next)

    s_parts = _sc_propagate(hp, src, dst, zblk)
    logits = pl.pallas_call(
        _final_body,
        out_shape=jax.ShapeDtypeStruct((NG, NL), f32),
    )(s_parts, hp, disb, b3, batch_pad, batch_b, Wc1, bc1, Wc2, bc2)
    return logits


# speedup vs baseline: 23.3680x; 1.3076x over previous
import functools

import jax
import jax.numpy as jnp
from jax import lax
from jax.experimental import pallas as pl
from jax.experimental.pallas import tpu as pltpu
from jax.experimental.pallas import tpu_sc as plsc

N = 10000
E = 640000
NPAD = 10240
NW = 32
EB = 125
NBLK = 160
NCH = 4
NBI = NBLK // NCH
RPT = NPAD // 16
XB = 80
XNB = 4
HID = 128
NG = 64
NL = 2


def _sc_precompute_body(dst_hbm, x_hbm, emb_hbm, zdeg_hbm, ones_hbm,
                        deg_out, h0_out, dst_v, ones_v, xv, erows, deg_sh, sem):
    c = lax.axis_index("c")
    s = lax.axis_index("s")
    w = s * 2 + c

    pltpu.sync_copy(dst_hbm.at[w], dst_v)
    pltpu.sync_copy(ones_hbm, ones_v)

    @pl.when(s == 0)
    def _():
        pltpu.sync_copy(zdeg_hbm, deg_sh)

    plsc.subcore_barrier()

    for jc in range(NCH):
        def deg_body(j, carry, jc=jc):
            pltpu.sync_copy(ones_v, deg_sh.at[dst_v.at[jc, j]], add=True)
            return carry

        lax.fori_loop(0, NBI, deg_body, 0)
    plsc.subcore_barrier()

    @pl.when(s == 0)
    def _():
        pltpu.sync_copy(deg_sh, deg_out.at[c])

    pltpu.sync_copy(x_hbm.at[w], xv)
    for jb in range(XNB):
        pltpu.async_copy(emb_hbm.at[xv.at[jb]], erows, sem).wait()
        pltpu.sync_copy(erows, h0_out.at[pl.ds(w * (XNB * XB) + jb * XB, XB)])


def _sc_propagate_body(hp_hbm, src_hbm, dst_hbm, zblk_hbm,
                       s_out, src_v, dst_v, rows0, rows1, acc_sh, sem):
    c = lax.axis_index("c")
    s = lax.axis_index("s")
    w = s * 2 + c

    pltpu.sync_copy(zblk_hbm, acc_sh.at[pl.ds(s * RPT, RPT)])
    plsc.subcore_barrier()

    def gather_start(j, buf):
        pltpu.async_copy(hp_hbm.at[src_v.at[j]], buf, sem)

    def gather_wait(buf):
        pltpu.make_async_copy(hp_hbm.at[src_v.at[0]], buf, sem).wait()

    for jc in range(NCH):
        pltpu.sync_copy(src_hbm.at[w, jc], src_v)
        pltpu.sync_copy(dst_hbm.at[w, jc], dst_v)
        gather_start(0, rows0)

        def body(i, carry):
            j = 2 * i
            gather_wait(rows0)
            gather_start(j + 1, rows1)
            pltpu.sync_copy(rows0, acc_sh.at[dst_v.at[j]], add=True)
            gather_wait(rows1)

            @pl.when(j + 2 < NBI)
            def _():
                gather_start(j + 2, rows0)

            pltpu.sync_copy(rows1, acc_sh.at[dst_v.at[j + 1]], add=True)
            return carry

        lax.fori_loop(0, NBI // 2, body, 0)
    plsc.subcore_barrier()
    pltpu.sync_copy(acc_sh.at[pl.ds(s * RPT, RPT)],
                    s_out.at[c, pl.ds(s * RPT, RPT)])


@functools.lru_cache(maxsize=None)
def _sc_kernels():
    mesh = plsc.VectorSubcoreMesh(core_axis_name="c", subcore_axis_name="s")
    precompute = functools.partial(
        pl.kernel,
        out_type=(
            jax.ShapeDtypeStruct((2, NPAD), jnp.float32),
            jax.ShapeDtypeStruct((NPAD, HID), jnp.float32),
        ),
        mesh=mesh,
        scratch_types=[
            pltpu.VMEM((NCH, NBI, EB), jnp.int32),
            pltpu.VMEM((EB,), jnp.float32),
            pltpu.VMEM((XNB, XB), jnp.int32),
            pltpu.VMEM((XB, HID), jnp.float32),
            pltpu.VMEM_SHARED((NPAD,), jnp.float32),
            pltpu.SemaphoreType.DMA,
        ],
    )(_sc_precompute_body)
    propagate = functools.partial(
        pl.kernel,
        out_type=jax.ShapeDtypeStruct((2, NPAD, HID), jnp.float32),
        mesh=mesh,
        scratch_types=[
            pltpu.VMEM((NBI, EB), jnp.int32),
            pltpu.VMEM((NBI, EB), jnp.int32),
            pltpu.VMEM((EB, HID), jnp.float32),
            pltpu.VMEM((EB, HID), jnp.float32),
            pltpu.VMEM_SHARED((NPAD, HID), jnp.float32),
            pltpu.SemaphoreType.DMA,
        ],
    )(_sc_propagate_body)
    return precompute, propagate


def _t0_body(deg_ref, h0_ref, w_ref, disb_ref, hp_ref):
    deg = deg_ref[0, :] + deg_ref[1, :] + 1.0
    dis = lax.rsqrt(jnp.maximum(deg, 1.0))
    disb = jnp.broadcast_to(dis[:, None], (NPAD, HID))
    disb_ref[...] = disb
    hp_ref[...] = disb * jnp.dot(h0_ref[...], w_ref[...],
                                 preferred_element_type=jnp.float32)


def _layer_body(s_ref, hp_ref, disb_ref, b_ref, w_ref, out_ref):
    tot = s_ref[0] + s_ref[1] + hp_ref[...]
    h = jnp.maximum(tot * disb_ref[...] + b_ref[...][None, :], 0.0)
    out_ref[...] = disb_ref[...] * jnp.dot(h, w_ref[...],
                                           preferred_element_type=jnp.float32)


def _final_body(s_ref, hp_ref, disb_ref, b_ref, batch_ref, batchb_ref,
                wc1_ref, bc1_ref, wc2_ref, bc2_ref, out_ref):
    tot = s_ref[0] + s_ref[1] + hp_ref[...]
    h = jnp.maximum(tot * disb_ref[...] + b_ref[...][None, :], 0.0)

    bt = batch_ref[...]
    gi = lax.broadcasted_iota(jnp.int32, (NG, NPAD), 0)
    m = (gi == bt[None, :]).astype(jnp.float32)
    counts = jnp.dot(m, jnp.ones((NPAD, 1), jnp.float32),
                     preferred_element_type=jnp.float32)
    sums = jnp.dot(m, h, preferred_element_type=jnp.float32)
    mean = sums / jnp.maximum(counts, 1.0)

    btb = batchb_ref[...]
    parts = [jnp.max(jnp.where(btb == g, h, 0.0), axis=0, keepdims=True)
             for g in range(NG)]
    maxs = jnp.concatenate(parts, axis=0)

    g1 = (jnp.dot(mean, wc1_ref[:HID, :], preferred_element_type=jnp.float32)
          + jnp.dot(maxs, wc1_ref[HID:, :], preferred_element_type=jnp.float32)
          + bc1_ref[...][None, :])
    g1 = jnp.maximum(g1, 0.0)
    out_ref[...] = (jnp.dot(g1, wc2_ref[...], preferred_element_type=jnp.float32)
                    + bc2_ref[...][None, :])


def kernel(x, edge_index, batch, emb_table,
           W0, b0, W1, b1, W2, b2, W3, b3, Wc1, bc1, Wc2, bc2):
    f32 = jnp.float32
    src = edge_index[0].astype(jnp.int32).reshape(NW, NCH, NBI, EB)
    dst = edge_index[1].astype(jnp.int32).reshape(NW, NCH, NBI, EB)
    xw = jnp.pad(x[:, 0].astype(jnp.int32), (0, NPAD - N)).reshape(NW, XNB, XB)
    batch_pad = jnp.pad(batch.astype(jnp.int32), (0, NPAD - N),
                        constant_values=NG)
    batch_b = jnp.broadcast_to(batch_pad[:, None], (NPAD, HID))

    zdeg = jnp.zeros((NPAD,), f32)
    zblk = jnp.zeros((RPT, HID), f32)
    onesb = jnp.ones((EB,), f32)

    _sc_precompute, _sc_propagate = _sc_kernels()
    deg2, h0 = _sc_precompute(dst, xw, emb_table, zdeg, onesb)

    disb, hp = pl.pallas_call(
        _t0_body,
        out_shape=(jax.ShapeDtypeStruct((NPAD, HID), f32),
                   jax.ShapeDtypeStruct((NPAD, HID), f32)),
    )(deg2, h0, W0)

    for (b_prev, w_next) in ((b0, W1), (b1, W2), (b2, W3)):
        s_parts = _sc_propagate(hp, src, dst, zblk)
        hp = pl.pallas_call(
            _layer_body,
            out_shape=jax.ShapeDtypeStruct((NPAD, HID), f32),
        )(s_parts, hp, disb, b_prev, w_next)

    s_parts = _sc_propagate(hp, src, dst, zblk)
    logits = pl.pallas_call(
        _final_body,
        out_shape=jax.ShapeDtypeStruct((NG, NL), f32),
    )(s_parts, hp, disb, b3, batch_pad, batch_b, Wc1, bc1, Wc2, bc2)
    return logits


# speedup vs baseline: 27.3911x; 1.1722x over previous
import functools

import jax
import jax.numpy as jnp
from jax import lax
from jax.experimental import pallas as pl
from jax.experimental.pallas import tpu as pltpu
from jax.experimental.pallas import tpu_sc as plsc

N = 10000
E = 640000
NPAD = 10240
NW = 32
EB = 125
NBLK = 160
NCH = 4
NBI = NBLK // NCH
RPT = NPAD // 16
XB = 80
XNB = 4
HID = 128
NG = 64
NL = 2


def _sc_precompute_body(dst_hbm, x_hbm, emb_hbm, zdeg_hbm, ones_hbm,
                        deg_out, h0_out, dst_v, ones_v, xv, erows, deg_sh, sem):
    c = lax.axis_index("c")
    s = lax.axis_index("s")
    w = s * 2 + c

    pltpu.sync_copy(dst_hbm.at[w], dst_v)
    pltpu.sync_copy(ones_hbm, ones_v)

    @pl.when(s == 0)
    def _():
        pltpu.sync_copy(zdeg_hbm, deg_sh)

    plsc.subcore_barrier()

    for jc in range(NCH):
        def deg_body(j, carry, jc=jc):
            pltpu.sync_copy(ones_v, deg_sh.at[dst_v.at[jc, j]], add=True)
            return carry

        lax.fori_loop(0, NBI, deg_body, 0)
    plsc.subcore_barrier()

    @pl.when(s == 0)
    def _():
        pltpu.sync_copy(deg_sh, deg_out.at[c])

    pltpu.sync_copy(x_hbm.at[w], xv)
    for jb in range(XNB):
        pltpu.async_copy(emb_hbm.at[xv.at[jb]], erows, sem).wait()
        pltpu.sync_copy(erows, h0_out.at[pl.ds(w * (XNB * XB) + jb * XB, XB)])


def _sc_propagate_body(hp_hbm, src_hbm, dst_hbm, zblk_hbm,
                       s_out, src_v, dst_v, rows0, rows1, acc_sh, sem0, sem1):
    c = lax.axis_index("c")
    s = lax.axis_index("s")
    w = s * 2 + c

    pltpu.sync_copy(zblk_hbm, acc_sh.at[pl.ds(s * RPT, RPT)])
    plsc.subcore_barrier()

    def gather_start(j, buf, sem):
        pltpu.async_copy(hp_hbm.at[src_v.at[j]], buf, sem)

    def gather_wait(buf, sem):
        pltpu.make_async_copy(hp_hbm.at[src_v.at[0]], buf, sem).wait()

    for jc in range(NCH):
        pltpu.sync_copy(src_hbm.at[w, jc], src_v)
        pltpu.sync_copy(dst_hbm.at[w, jc], dst_v)
        gather_start(0, rows0, sem0)
        gather_start(1, rows1, sem1)

        def body(i, carry):
            j = 2 * i
            gather_wait(rows0, sem0)
            pltpu.sync_copy(rows0, acc_sh.at[dst_v.at[j]], add=True)

            @pl.when(j + 2 < NBI)
            def _():
                gather_start(j + 2, rows0, sem0)

            gather_wait(rows1, sem1)
            pltpu.sync_copy(rows1, acc_sh.at[dst_v.at[j + 1]], add=True)

            @pl.when(j + 3 < NBI)
            def _():
                gather_start(j + 3, rows1, sem1)

            return carry

        lax.fori_loop(0, NBI // 2, body, 0)
    plsc.subcore_barrier()
    pltpu.sync_copy(acc_sh.at[pl.ds(s * RPT, RPT)],
                    s_out.at[c, pl.ds(s * RPT, RPT)])


@functools.lru_cache(maxsize=None)
def _sc_kernels():
    mesh = plsc.VectorSubcoreMesh(core_axis_name="c", subcore_axis_name="s")
    precompute = functools.partial(
        pl.kernel,
        out_type=(
            jax.ShapeDtypeStruct((2, NPAD), jnp.float32),
            jax.ShapeDtypeStruct((NPAD, HID), jnp.float32),
        ),
        mesh=mesh,
        scratch_types=[
            pltpu.VMEM((NCH, NBI, EB), jnp.int32),
            pltpu.VMEM((EB,), jnp.float32),
            pltpu.VMEM((XNB, XB), jnp.int32),
            pltpu.VMEM((XB, HID), jnp.float32),
            pltpu.VMEM_SHARED((NPAD,), jnp.float32),
            pltpu.SemaphoreType.DMA,
        ],
    )(_sc_precompute_body)
    propagate = functools.partial(
        pl.kernel,
        out_type=jax.ShapeDtypeStruct((2, NPAD, HID), jnp.float32),
        mesh=mesh,
        scratch_types=[
            pltpu.VMEM((NBI, EB), jnp.int32),
            pltpu.VMEM((NBI, EB), jnp.int32),
            pltpu.VMEM((EB, HID), jnp.float32),
            pltpu.VMEM((EB, HID), jnp.float32),
            pltpu.VMEM_SHARED((NPAD, HID), jnp.float32),
            pltpu.SemaphoreType.DMA,
            pltpu.SemaphoreType.DMA,
        ],
    )(_sc_propagate_body)
    return precompute, propagate


def _t0_body(deg_ref, h0_ref, w_ref, disb_ref, hp_ref):
    deg = deg_ref[0, :] + deg_ref[1, :] + 1.0
    dis = lax.rsqrt(jnp.maximum(deg, 1.0))
    disb = jnp.broadcast_to(dis[:, None], (NPAD, HID))
    disb_ref[...] = disb
    hp_ref[...] = disb * jnp.dot(h0_ref[...], w_ref[...],
                                 preferred_element_type=jnp.float32)


def _layer_body(s_ref, hp_ref, disb_ref, b_ref, w_ref, out_ref):
    tot = s_ref[0] + s_ref[1] + hp_ref[...]
    h = jnp.maximum(tot * disb_ref[...] + b_ref[...][None, :], 0.0)
    out_ref[...] = disb_ref[...] * jnp.dot(h, w_ref[...],
                                           preferred_element_type=jnp.float32)


def _final_body(s_ref, hp_ref, disb_ref, b_ref, batch_ref, batchb_ref,
                wc1_ref, bc1_ref, wc2_ref, bc2_ref, out_ref):
    tot = s_ref[0] + s_ref[1] + hp_ref[...]
    h = jnp.maximum(tot * disb_ref[...] + b_ref[...][None, :], 0.0)

    bt = batch_ref[...]
    gi = lax.broadcasted_iota(jnp.int32, (NG, NPAD), 0)
    m = (gi == bt[None, :]).astype(jnp.float32)
    counts = jnp.dot(m, jnp.ones((NPAD, 1), jnp.float32),
                     preferred_element_type=jnp.float32)
    sums = jnp.dot(m, h, preferred_element_type=jnp.float32)
    mean = sums / jnp.maximum(counts, 1.0)

    btb = batchb_ref[...]
    parts = [jnp.max(jnp.where(btb == g, h, 0.0), axis=0, keepdims=True)
             for g in range(NG)]
    maxs = jnp.concatenate(parts, axis=0)

    g1 = (jnp.dot(mean, wc1_ref[:HID, :], preferred_element_type=jnp.float32)
          + jnp.dot(maxs, wc1_ref[HID:, :], preferred_element_type=jnp.float32)
          + bc1_ref[...][None, :])
    g1 = jnp.maximum(g1, 0.0)
    out_ref[...] = (jnp.dot(g1, wc2_ref[...], preferred_element_type=jnp.float32)
                    + bc2_ref[...][None, :])


def kernel(x, edge_index, batch, emb_table,
           W0, b0, W1, b1, W2, b2, W3, b3, Wc1, bc1, Wc2, bc2):
    f32 = jnp.float32
    src = edge_index[0].astype(jnp.int32).reshape(NW, NCH, NBI, EB)
    dst = edge_index[1].astype(jnp.int32).reshape(NW, NCH, NBI, EB)
    xw = jnp.pad(x[:, 0].astype(jnp.int32), (0, NPAD - N)).reshape(NW, XNB, XB)
    batch_pad = jnp.pad(batch.astype(jnp.int32), (0, NPAD - N),
                        constant_values=NG)
    batch_b = jnp.broadcast_to(batch_pad[:, None], (NPAD, HID))

    zdeg = jnp.zeros((NPAD,), f32)
    zblk = jnp.zeros((RPT, HID), f32)
    onesb = jnp.ones((EB,), f32)

    _sc_precompute, _sc_propagate = _sc_kernels()
    deg2, h0 = _sc_precompute(dst, xw, emb_table, zdeg, onesb)

    disb, hp = pl.pallas_call(
        _t0_body,
        out_shape=(jax.ShapeDtypeStruct((NPAD, HID), f32),
                   jax.ShapeDtypeStruct((NPAD, HID), f32)),
    )(deg2, h0, W0)

    for (b_prev, w_next) in ((b0, W1), (b1, W2), (b2, W3)):
        s_parts = _sc_propagate(hp, src, dst, zblk)
        hp = pl.pallas_call(
            _layer_body,
            out_shape=jax.ShapeDtypeStruct((NPAD, HID), f32),
        )(s_parts, hp, disb, b_prev, w_next)

    s_parts = _sc_propagate(hp, src, dst, zblk)
    logits = pl.pallas_call(
        _final_body,
        out_shape=jax.ShapeDtypeStruct((NG, NL), f32),
    )(s_parts, hp, disb, b3, batch_pad, batch_b, Wc1, bc1, Wc2, bc2)
    return logits


# speedup vs baseline: 29.1316x; 1.0635x over previous
import functools

import jax
import jax.numpy as jnp
from jax import lax
from jax.experimental import pallas as pl
from jax.experimental.pallas import tpu as pltpu
from jax.experimental.pallas import tpu_sc as plsc

N = 10000
E = 640000
NPAD = 10240
NW = 32
EB = 80
NBLK = 250
NCH = 5
NBI = NBLK // NCH
RPT = NPAD // 16
XB = 80
XNB = 4
HID = 128
NG = 64
NL = 2


def _sc_precompute_body(dst_hbm, x_hbm, emb_hbm, zdeg_hbm, ones_hbm,
                        deg_out, h0_out, dst_v, ones_v, xv, erows, deg_sh, sem):
    c = lax.axis_index("c")
    s = lax.axis_index("s")
    w = s * 2 + c

    pltpu.sync_copy(dst_hbm.at[w], dst_v)
    pltpu.sync_copy(ones_hbm, ones_v)

    @pl.when(s == 0)
    def _():
        pltpu.sync_copy(zdeg_hbm, deg_sh)

    plsc.subcore_barrier()

    for jc in range(NCH):
        def deg_body(j, carry, jc=jc):
            pltpu.sync_copy(ones_v, deg_sh.at[dst_v.at[jc, j]], add=True)
            return carry

        lax.fori_loop(0, NBI, deg_body, 0)
    plsc.subcore_barrier()

    @pl.when(s == 0)
    def _():
        pltpu.sync_copy(deg_sh, deg_out.at[c])

    pltpu.sync_copy(x_hbm.at[w], xv)
    for jb in range(XNB):
        pltpu.async_copy(emb_hbm.at[xv.at[jb]], erows, sem).wait()
        pltpu.sync_copy(erows, h0_out.at[pl.ds(w * (XNB * XB) + jb * XB, XB)])


def _sc_propagate_body(hp_hbm, src_hbm, dst_hbm, zblk_hbm,
                       s_out, src_v, dst_v, rows0, rows1, rows2, acc_sh,
                       sem0, sem1, sem2):
    c = lax.axis_index("c")
    s = lax.axis_index("s")
    w = s * 2 + c

    pltpu.sync_copy(zblk_hbm, acc_sh.at[pl.ds(s * RPT, RPT)])
    plsc.subcore_barrier()

    def gather_start(j, buf, sem):
        pltpu.async_copy(hp_hbm.at[src_v.at[j]], buf, sem)

    def gather_wait(buf, sem):
        pltpu.make_async_copy(hp_hbm.at[src_v.at[0]], buf, sem).wait()

    bufs = ((rows0, sem0), (rows1, sem1), (rows2, sem2))

    def step(j, b, guarded=True):
        buf, sem = bufs[b]
        gather_wait(buf, sem)
        pltpu.sync_copy(buf, acc_sh.at[dst_v.at[j]], add=True)
        if guarded:
            @pl.when(j + 3 < NBI)
            def _():
                gather_start(j + 3, buf, sem)

    for jc in range(NCH):
        pltpu.sync_copy(src_hbm.at[w, jc], src_v)
        pltpu.sync_copy(dst_hbm.at[w, jc], dst_v)
        for b in range(3):
            gather_start(b, *bufs[b])

        def body(i, carry):
            j = 3 * i
            step(j, 0)
            step(j + 1, 1)
            step(j + 2, 2)
            return carry

        nfull = NBI // 3
        lax.fori_loop(0, nfull, body, 0)
        for j in range(3 * nfull, NBI):
            step(j, j % 3, guarded=False)
    plsc.subcore_barrier()
    pltpu.sync_copy(acc_sh.at[pl.ds(s * RPT, RPT)],
                    s_out.at[c, pl.ds(s * RPT, RPT)])


@functools.lru_cache(maxsize=None)
def _sc_kernels():
    mesh = plsc.VectorSubcoreMesh(core_axis_name="c", subcore_axis_name="s")
    precompute = functools.partial(
        pl.kernel,
        out_type=(
            jax.ShapeDtypeStruct((2, NPAD), jnp.float32),
            jax.ShapeDtypeStruct((NPAD, HID), jnp.float32),
        ),
        mesh=mesh,
        scratch_types=[
            pltpu.VMEM((NCH, NBI, EB), jnp.int32),
            pltpu.VMEM((EB,), jnp.float32),
            pltpu.VMEM((XNB, XB), jnp.int32),
            pltpu.VMEM((XB, HID), jnp.float32),
            pltpu.VMEM_SHARED((NPAD,), jnp.float32),
            pltpu.SemaphoreType.DMA,
        ],
    )(_sc_precompute_body)
    propagate = functools.partial(
        pl.kernel,
        out_type=jax.ShapeDtypeStruct((2, NPAD, HID), jnp.float32),
        mesh=mesh,
        scratch_types=[
            pltpu.VMEM((NBI, EB), jnp.int32),
            pltpu.VMEM((NBI, EB), jnp.int32),
            pltpu.VMEM((EB, HID), jnp.float32),
            pltpu.VMEM((EB, HID), jnp.float32),
            pltpu.VMEM((EB, HID), jnp.float32),
            pltpu.VMEM_SHARED((NPAD, HID), jnp.float32),
            pltpu.SemaphoreType.DMA,
            pltpu.SemaphoreType.DMA,
            pltpu.SemaphoreType.DMA,
        ],
    )(_sc_propagate_body)
    return precompute, propagate


def _t0_body(deg_ref, h0_ref, w_ref, disb_ref, hp_ref):
    deg = deg_ref[0, :] + deg_ref[1, :] + 1.0
    dis = lax.rsqrt(jnp.maximum(deg, 1.0))
    disb = jnp.broadcast_to(dis[:, None], (NPAD, HID))
    disb_ref[...] = disb
    hp_ref[...] = disb * jnp.dot(h0_ref[...], w_ref[...],
                                 preferred_element_type=jnp.float32)


def _layer_body(s_ref, hp_ref, disb_ref, b_ref, w_ref, out_ref):
    tot = s_ref[0] + s_ref[1] + hp_ref[...]
    h = jnp.maximum(tot * disb_ref[...] + b_ref[...][None, :], 0.0)
    out_ref[...] = disb_ref[...] * jnp.dot(h, w_ref[...],
                                           preferred_element_type=jnp.float32)


def _final_body(s_ref, hp_ref, disb_ref, b_ref, batch_ref, batchb_ref,
                wc1_ref, bc1_ref, wc2_ref, bc2_ref, out_ref):
    tot = s_ref[0] + s_ref[1] + hp_ref[...]
    h = jnp.maximum(tot * disb_ref[...] + b_ref[...][None, :], 0.0)

    bt = batch_ref[...]
    gi = lax.broadcasted_iota(jnp.int32, (NG, NPAD), 0)
    m = (gi == bt[None, :]).astype(jnp.float32)
    counts = jnp.dot(m, jnp.ones((NPAD, 1), jnp.float32),
                     preferred_element_type=jnp.float32)
    sums = jnp.dot(m, h, preferred_element_type=jnp.float32)
    mean = sums / jnp.maximum(counts, 1.0)

    btb = batchb_ref[...]
    parts = [jnp.max(jnp.where(btb == g, h, 0.0), axis=0, keepdims=True)
             for g in range(NG)]
    maxs = jnp.concatenate(parts, axis=0)

    g1 = (jnp.dot(mean, wc1_ref[:HID, :], preferred_element_type=jnp.float32)
          + jnp.dot(maxs, wc1_ref[HID:, :], preferred_element_type=jnp.float32)
          + bc1_ref[...][None, :])
    g1 = jnp.maximum(g1, 0.0)
    out_ref[...] = (jnp.dot(g1, wc2_ref[...], preferred_element_type=jnp.float32)
                    + bc2_ref[...][None, :])


def kernel(x, edge_index, batch, emb_table,
           W0, b0, W1, b1, W2, b2, W3, b3, Wc1, bc1, Wc2, bc2):
    f32 = jnp.float32
    src = edge_index[0].astype(jnp.int32).reshape(NW, NCH, NBI, EB)
    dst = edge_index[1].astype(jnp.int32).reshape(NW, NCH, NBI, EB)
    xw = jnp.pad(x[:, 0].astype(jnp.int32), (0, NPAD - N)).reshape(NW, XNB, XB)
    batch_pad = jnp.pad(batch.astype(jnp.int32), (0, NPAD - N),
                        constant_values=NG)
    batch_b = jnp.broadcast_to(batch_pad[:, None], (NPAD, HID))

    zdeg = jnp.zeros((NPAD,), f32)
    zblk = jnp.zeros((RPT, HID), f32)
    onesb = jnp.ones((EB,), f32)

    _sc_precompute, _sc_propagate = _sc_kernels()
    deg2, h0 = _sc_precompute(dst, xw, emb_table, zdeg, onesb)

    disb, hp = pl.pallas_call(
        _t0_body,
        out_shape=(jax.ShapeDtypeStruct((NPAD, HID), f32),
                   jax.ShapeDtypeStruct((NPAD, HID), f32)),
    )(deg2, h0, W0)

    for (b_prev, w_next) in ((b0, W1), (b1, W2), (b2, W3)):
        s_parts = _sc_propagate(hp, src, dst, zblk)
        hp = pl.pallas_call(
            _layer_body,
            out_shape=jax.ShapeDtypeStruct((NPAD, HID), f32),
        )(s_parts, hp, disb, b_prev, w_next)

    s_parts = _sc_propagate(hp, src, dst, zblk)
    logits = pl.pallas_call(
        _final_body,
        out_shape=jax.ShapeDtypeStruct((NG, NL), f32),
    )(s_parts, hp, disb, b3, batch_pad, batch_b, Wc1, bc1, Wc2, bc2)
    return logits


# speedup vs baseline: 29.4278x; 1.0102x over previous
import functools

import jax
import jax.numpy as jnp
from jax import lax
from jax.experimental import pallas as pl
from jax.experimental.pallas import tpu as pltpu
from jax.experimental.pallas import tpu_sc as plsc

N = 10000
E = 640000
NPAD = 10240
NW = 32
EB = 50
NBLK = 400
NCH = 5
NBI = NBLK // NCH
RPT = NPAD // 16
XB = 80
XNB = 4
HID = 128
NG = 64
NL = 2


def _sc_precompute_body(dst_hbm, x_hbm, emb_hbm, zdeg_hbm, ones_hbm,
                        deg_out, h0_out, dst_v, ones_v, xv, erows, deg_sh, sem):
    c = lax.axis_index("c")
    s = lax.axis_index("s")
    w = s * 2 + c

    pltpu.sync_copy(dst_hbm.at[w], dst_v)
    pltpu.sync_copy(ones_hbm, ones_v)

    @pl.when(s == 0)
    def _():
        pltpu.sync_copy(zdeg_hbm, deg_sh)

    plsc.subcore_barrier()

    for jc in range(NCH):
        def deg_body(i, carry, jc=jc):
            for u in range(4):
                pltpu.async_copy(ones_v, deg_sh.at[dst_v.at[jc, 4 * i + u]],
                                 sem, add=True)
            for u in range(4):
                pltpu.make_async_copy(ones_v, deg_sh.at[dst_v.at[jc, 0]],
                                      sem).wait()
            return carry

        lax.fori_loop(0, NBI // 4, deg_body, 0)
    plsc.subcore_barrier()

    @pl.when(s == 0)
    def _():
        pltpu.sync_copy(deg_sh, deg_out.at[c])

    pltpu.sync_copy(x_hbm.at[w], xv)
    for jb in range(XNB):
        pltpu.async_copy(emb_hbm.at[xv.at[jb]], erows, sem).wait()
        pltpu.sync_copy(erows, h0_out.at[pl.ds(w * (XNB * XB) + jb * XB, XB)])


NBUF = 4


def _sc_propagate_body(hp_hbm, src_hbm, dst_hbm, zblk_hbm,
                       s_out, src_v, dst_v, rows0, rows1, rows2, rows3,
                       acc_sh, sem0, sem1, sem2, sem3):
    c = lax.axis_index("c")
    s = lax.axis_index("s")
    w = s * 2 + c

    pltpu.sync_copy(zblk_hbm, acc_sh.at[pl.ds(s * RPT, RPT)])
    plsc.subcore_barrier()

    def gather_start(j, buf, sem):
        pltpu.async_copy(hp_hbm.at[src_v.at[j]], buf, sem)

    def gather_wait(buf, sem):
        pltpu.make_async_copy(hp_hbm.at[src_v.at[0]], buf, sem).wait()

    bufs = ((rows0, sem0), (rows1, sem1), (rows2, sem2), (rows3, sem3))

    def step(j, b):
        buf, sem = bufs[b]
        gather_wait(buf, sem)
        pltpu.sync_copy(buf, acc_sh.at[dst_v.at[j]], add=True)

        @pl.when(j + NBUF < NBI)
        def _():
            gather_start(j + NBUF, buf, sem)

    for jc in range(NCH):
        pltpu.sync_copy(src_hbm.at[w, jc], src_v)
        pltpu.sync_copy(dst_hbm.at[w, jc], dst_v)
        for b in range(NBUF):
            gather_start(b, *bufs[b])

        def body(i, carry):
            j = NBUF * i
            for b in range(NBUF):
                step(j + b, b)
            return carry

        lax.fori_loop(0, NBI // NBUF, body, 0)
    plsc.subcore_barrier()
    pltpu.sync_copy(acc_sh.at[pl.ds(s * RPT, RPT)],
                    s_out.at[c, pl.ds(s * RPT, RPT)])


@functools.lru_cache(maxsize=None)
def _sc_kernels():
    mesh = plsc.VectorSubcoreMesh(core_axis_name="c", subcore_axis_name="s")
    precompute = functools.partial(
        pl.kernel,
        out_type=(
            jax.ShapeDtypeStruct((2, NPAD), jnp.float32),
            jax.ShapeDtypeStruct((NPAD, HID), jnp.float32),
        ),
        mesh=mesh,
        scratch_types=[
            pltpu.VMEM((NCH, NBI, EB), jnp.int32),
            pltpu.VMEM((EB,), jnp.float32),
            pltpu.VMEM((XNB, XB), jnp.int32),
            pltpu.VMEM((XB, HID), jnp.float32),
            pltpu.VMEM_SHARED((NPAD,), jnp.float32),
            pltpu.SemaphoreType.DMA,
        ],
    )(_sc_precompute_body)
    propagate = functools.partial(
        pl.kernel,
        out_type=jax.ShapeDtypeStruct((2, NPAD, HID), jnp.float32),
        mesh=mesh,
        scratch_types=[
            pltpu.VMEM((NBI, EB), jnp.int32),
            pltpu.VMEM((NBI, EB), jnp.int32),
            pltpu.VMEM((EB, HID), jnp.float32),
            pltpu.VMEM((EB, HID), jnp.float32),
            pltpu.VMEM((EB, HID), jnp.float32),
            pltpu.VMEM((EB, HID), jnp.float32),
            pltpu.VMEM_SHARED((NPAD, HID), jnp.float32),
            pltpu.SemaphoreType.DMA,
            pltpu.SemaphoreType.DMA,
            pltpu.SemaphoreType.DMA,
            pltpu.SemaphoreType.DMA,
        ],
    )(_sc_propagate_body)
    return precompute, propagate


def _t0_body(deg_ref, h0_ref, w_ref, disb_ref, hp_ref):
    deg = deg_ref[0, :] + deg_ref[1, :] + 1.0
    dis = lax.rsqrt(jnp.maximum(deg, 1.0))
    disb = jnp.broadcast_to(dis[:, None], (NPAD, HID))
    disb_ref[...] = disb
    hp_ref[...] = disb * jnp.dot(h0_ref[...], w_ref[...],
                                 preferred_element_type=jnp.float32)


def _layer_body(s_ref, hp_ref, disb_ref, b_ref, w_ref, out_ref):
    tot = s_ref[0] + s_ref[1] + hp_ref[...]
    h = jnp.maximum(tot * disb_ref[...] + b_ref[...][None, :], 0.0)
    out_ref[...] = disb_ref[...] * jnp.dot(h, w_ref[...],
                                           preferred_element_type=jnp.float32)


def _final_body(s_ref, hp_ref, disb_ref, b_ref, batch_ref, batchb_ref,
                wc1_ref, bc1_ref, wc2_ref, bc2_ref, out_ref):
    tot = s_ref[0] + s_ref[1] + hp_ref[...]
    h = jnp.maximum(tot * disb_ref[...] + b_ref[...][None, :], 0.0)

    bt = batch_ref[...]
    gi = lax.broadcasted_iota(jnp.int32, (NG, NPAD), 0)
    m = (gi == bt[None, :]).astype(jnp.float32)
    counts = jnp.dot(m, jnp.ones((NPAD, 1), jnp.float32),
                     preferred_element_type=jnp.float32)
    sums = jnp.dot(m, h, preferred_element_type=jnp.float32)
    mean = sums / jnp.maximum(counts, 1.0)

    btb = batchb_ref[...]
    parts = [jnp.max(jnp.where(btb == g, h, 0.0), axis=0, keepdims=True)
             for g in range(NG)]
    maxs = jnp.concatenate(parts, axis=0)

    g1 = (jnp.dot(mean, wc1_ref[:HID, :], preferred_element_type=jnp.float32)
          + jnp.dot(maxs, wc1_ref[HID:, :], preferred_element_type=jnp.float32)
          + bc1_ref[...][None, :])
    g1 = jnp.maximum(g1, 0.0)
    out_ref[...] = (jnp.dot(g1, wc2_ref[...], preferred_element_type=jnp.float32)
                    + bc2_ref[...][None, :])


def kernel(x, edge_index, batch, emb_table,
           W0, b0, W1, b1, W2, b2, W3, b3, Wc1, bc1, Wc2, bc2):
    f32 = jnp.float32
    src = edge_index[0].astype(jnp.int32).reshape(NW, NCH, NBI, EB)
    dst = edge_index[1].astype(jnp.int32).reshape(NW, NCH, NBI, EB)
    xw = jnp.pad(x[:, 0].astype(jnp.int32), (0, NPAD - N)).reshape(NW, XNB, XB)
    batch_pad = jnp.pad(batch.astype(jnp.int32), (0, NPAD - N),
                        constant_values=NG)
    batch_b = jnp.broadcast_to(batch_pad[:, None], (NPAD, HID))

    zdeg = jnp.zeros((NPAD,), f32)
    zblk = jnp.zeros((RPT, HID), f32)
    onesb = jnp.ones((EB,), f32)

    _sc_precompute, _sc_propagate = _sc_kernels()
    deg2, h0 = _sc_precompute(dst, xw, emb_table, zdeg, onesb)

    disb, hp = pl.pallas_call(
        _t0_body,
        out_shape=(jax.ShapeDtypeStruct((NPAD, HID), f32),
                   jax.ShapeDtypeStruct((NPAD, HID), f32)),
    )(deg2, h0, W0)

    for (b_prev, w_next) in ((b0, W1), (b1, W2), (b2, W3)):
        s_parts = _sc_propagate(hp, src, dst, zblk)
        hp = pl.pallas_call(
            _layer_body,
            out_shape=jax.ShapeDtypeStruct((NPAD, HID), f32),
        )(s_parts, hp, disb, b_prev, w_next)

    s_parts = _sc_propagate(hp, src, dst, zblk)
    logits = pl.pallas_call(
        _final_body,
        out_shape=jax.ShapeDtypeStruct((NG, NL), f32),
    )(s_parts, hp, disb, b3, batch_pad, batch_b, Wc1, bc1, Wc2, bc2)
    return logits


# speedup vs baseline: 29.9156x; 1.0166x over previous
import functools

import jax
import jax.numpy as jnp
from jax import lax
from jax.experimental import pallas as pl
from jax.experimental.pallas import tpu as pltpu
from jax.experimental.pallas import tpu_sc as plsc

N = 10000
E = 640000
NPAD = 10240
NW = 32
EB = 50
NBLK = 400
NCH = 5
NBI = NBLK // NCH
RPT = NPAD // 16
XB = 80
XNB = 4
HID = 128
NG = 64
NL = 2


def _sc_precompute_body(dst_hbm, x_hbm, emb_hbm, zdeg_hbm, ones_hbm,
                        deg_out, h0_out, dst_v, ones_v, xv, erows, deg_sh, sem):
    c = lax.axis_index("c")
    s = lax.axis_index("s")
    w = s * 2 + c

    pltpu.sync_copy(dst_hbm.at[w], dst_v)
    pltpu.sync_copy(ones_hbm, ones_v)

    @pl.when(s == 0)
    def _():
        pltpu.sync_copy(zdeg_hbm, deg_sh)

    plsc.subcore_barrier()

    for jc in range(NCH):
        def deg_body(i, carry, jc=jc):
            for u in range(4):
                pltpu.async_copy(ones_v, deg_sh.at[dst_v.at[jc, 4 * i + u]],
                                 sem, add=True)
            for u in range(4):
                pltpu.make_async_copy(ones_v, deg_sh.at[dst_v.at[jc, 0]],
                                      sem).wait()
            return carry

        lax.fori_loop(0, NBI // 4, deg_body, 0)
    plsc.subcore_barrier()

    @pl.when(s == 0)
    def _():
        pltpu.sync_copy(deg_sh, deg_out.at[c])

    pltpu.sync_copy(x_hbm.at[w], xv)
    for jb in range(XNB):
        pltpu.async_copy(emb_hbm.at[xv.at[jb]], erows, sem).wait()
        pltpu.sync_copy(erows, h0_out.at[pl.ds(w * (XNB * XB) + jb * XB, XB)])


NBUF = 4


def _sc_propagate_body(hp_hbm, src_hbm, dst_hbm,
                       s_out, src_v, dst_v, rows0, rows1, rows2, rows3,
                       acc_sh, sem0, sem1, sem2, sem3):
    c = lax.axis_index("c")
    s = lax.axis_index("s")
    w = s * 2 + c

    z16 = jnp.zeros((16,), jnp.float32)

    def zrow(r, carry):
        for k in range(HID // 16):
            rows0[r, pl.ds(16 * k, 16)] = z16
        return carry

    lax.fori_loop(0, EB, zrow, 0)
    nz = RPT // EB
    for k in range(nz):
        pltpu.sync_copy(rows0, acc_sh.at[pl.ds(s * RPT + k * EB, EB)])
    rem = RPT - nz * EB
    if rem:
        pltpu.sync_copy(rows0.at[pl.ds(0, rem)],
                        acc_sh.at[pl.ds(s * RPT + nz * EB, rem)])
    plsc.subcore_barrier()

    def gather_start(j, buf, sem):
        pltpu.async_copy(hp_hbm.at[src_v.at[j]], buf, sem)

    def gather_wait(buf, sem):
        pltpu.make_async_copy(hp_hbm.at[src_v.at[0]], buf, sem).wait()

    bufs = ((rows0, sem0), (rows1, sem1), (rows2, sem2), (rows3, sem3))

    def step(j, b):
        buf, sem = bufs[b]
        gather_wait(buf, sem)
        pltpu.sync_copy(buf, acc_sh.at[dst_v.at[j]], add=True)

        @pl.when(j + NBUF < NBI)
        def _():
            gather_start(j + NBUF, buf, sem)

    for jc in range(NCH):
        pltpu.sync_copy(src_hbm.at[w, jc], src_v)
        pltpu.sync_copy(dst_hbm.at[w, jc], dst_v)
        for b in range(NBUF):
            gather_start(b, *bufs[b])

        def body(i, carry):
            j = NBUF * i
            for b in range(NBUF):
                step(j + b, b)
            return carry

        lax.fori_loop(0, NBI // NBUF, body, 0)
    plsc.subcore_barrier()
    pltpu.sync_copy(acc_sh.at[pl.ds(s * RPT, RPT)],
                    s_out.at[c, pl.ds(s * RPT, RPT)])


@functools.lru_cache(maxsize=None)
def _sc_kernels():
    mesh = plsc.VectorSubcoreMesh(core_axis_name="c", subcore_axis_name="s")
    precompute = functools.partial(
        pl.kernel,
        out_type=(
            jax.ShapeDtypeStruct((2, NPAD), jnp.float32),
            jax.ShapeDtypeStruct((NPAD, HID), jnp.float32),
        ),
        mesh=mesh,
        scratch_types=[
            pltpu.VMEM((NCH, NBI, EB), jnp.int32),
            pltpu.VMEM((EB,), jnp.float32),
            pltpu.VMEM((XNB, XB), jnp.int32),
            pltpu.VMEM((XB, HID), jnp.float32),
            pltpu.VMEM_SHARED((NPAD,), jnp.float32),
            pltpu.SemaphoreType.DMA,
        ],
    )(_sc_precompute_body)
    propagate = functools.partial(
        pl.kernel,
        out_type=jax.ShapeDtypeStruct((2, NPAD, HID), jnp.float32),
        mesh=mesh,
        scratch_types=[
            pltpu.VMEM((NBI, EB), jnp.int32),
            pltpu.VMEM((NBI, EB), jnp.int32),
            pltpu.VMEM((EB, HID), jnp.float32),
            pltpu.VMEM((EB, HID), jnp.float32),
            pltpu.VMEM((EB, HID), jnp.float32),
            pltpu.VMEM((EB, HID), jnp.float32),
            pltpu.VMEM_SHARED((NPAD, HID), jnp.float32),
            pltpu.SemaphoreType.DMA,
            pltpu.SemaphoreType.DMA,
            pltpu.SemaphoreType.DMA,
            pltpu.SemaphoreType.DMA,
        ],
    )(_sc_propagate_body)
    return precompute, propagate


def _t0_body(deg_ref, h0_ref, w_ref, disb_ref, hp_ref):
    deg = deg_ref[0, :] + deg_ref[1, :] + 1.0
    dis = lax.rsqrt(jnp.maximum(deg, 1.0))
    disb = jnp.broadcast_to(dis[:, None], (NPAD, HID))
    disb_ref[...] = disb
    hp_ref[...] = disb * jnp.dot(h0_ref[...], w_ref[...],
                                 preferred_element_type=jnp.float32)


def _layer_body(s_ref, hp_ref, disb_ref, b_ref, w_ref, out_ref):
    tot = s_ref[0] + s_ref[1] + hp_ref[...]
    h = jnp.maximum(tot * disb_ref[...] + b_ref[...][None, :], 0.0)
    out_ref[...] = disb_ref[...] * jnp.dot(h, w_ref[...],
                                           preferred_element_type=jnp.float32)


def _final_body(s_ref, hp_ref, disb_ref, b_ref, batch_ref, batchb_ref,
                wc1_ref, bc1_ref, wc2_ref, bc2_ref, out_ref):
    tot = s_ref[0] + s_ref[1] + hp_ref[...]
    h = jnp.maximum(tot * disb_ref[...] + b_ref[...][None, :], 0.0)

    bt = batch_ref[...]
    gi = lax.broadcasted_iota(jnp.int32, (NG, NPAD), 0)
    m = (gi == bt[None, :]).astype(jnp.float32)
    counts = jnp.dot(m, jnp.ones((NPAD, 1), jnp.float32),
                     preferred_element_type=jnp.float32)
    sums = jnp.dot(m, h, preferred_element_type=jnp.float32)
    mean = sums / jnp.maximum(counts, 1.0)

    btb = batchb_ref[...]
    parts = [jnp.max(jnp.where(btb == g, h, 0.0), axis=0, keepdims=True)
             for g in range(NG)]
    maxs = jnp.concatenate(parts, axis=0)

    g1 = (jnp.dot(mean, wc1_ref[:HID, :], preferred_element_type=jnp.float32)
          + jnp.dot(maxs, wc1_ref[HID:, :], preferred_element_type=jnp.float32)
          + bc1_ref[...][None, :])
    g1 = jnp.maximum(g1, 0.0)
    out_ref[...] = (jnp.dot(g1, wc2_ref[...], preferred_element_type=jnp.float32)
                    + bc2_ref[...][None, :])


def kernel(x, edge_index, batch, emb_table,
           W0, b0, W1, b1, W2, b2, W3, b3, Wc1, bc1, Wc2, bc2):
    f32 = jnp.float32
    src = edge_index[0].astype(jnp.int32).reshape(NW, NCH, NBI, EB)
    dst = edge_index[1].astype(jnp.int32).reshape(NW, NCH, NBI, EB)
    xw = jnp.pad(x[:, 0].astype(jnp.int32), (0, NPAD - N)).reshape(NW, XNB, XB)
    batch_pad = jnp.pad(batch.astype(jnp.int32), (0, NPAD - N),
                        constant_values=NG)
    batch_b = jnp.broadcast_to(batch_pad[:, None], (NPAD, HID))

    zdeg = jnp.zeros((NPAD,), f32)
    onesb = jnp.ones((EB,), f32)

    _sc_precompute, _sc_propagate = _sc_kernels()
    deg2, h0 = _sc_precompute(dst, xw, emb_table, zdeg, onesb)

    disb, hp = pl.pallas_call(
        _t0_body,
        out_shape=(jax.ShapeDtypeStruct((NPAD, HID), f32),
                   jax.ShapeDtypeStruct((NPAD, HID), f32)),
    )(deg2, h0, W0)

    for (b_prev, w_next) in ((b0, W1), (b1, W2), (b2, W3)):
        s_parts = _sc_propagate(hp, src, dst)
        hp = pl.pallas_call(
            _layer_body,
            out_shape=jax.ShapeDtypeStruct((NPAD, HID), f32),
        )(s_parts, hp, disb, b_prev, w_next)

    s_parts = _sc_propagate(hp, src, dst)
    logits = pl.pallas_call(
        _final_body,
        out_shape=jax.ShapeDtypeStruct((NG, NL), f32),
    )(s_parts, hp, disb, b3, batch_pad, batch_b, Wc1, bc1, Wc2, bc2)
    return logits
